# baseline (speedup 1.0000x reference)
import functools

import jax
import jax.numpy as jnp
from jax import lax
from jax.experimental import pallas as pl
from jax.experimental.pallas import tpu as pltpu
from jax.experimental.pallas import tpu_sc as plsc

N_ROWS = 8192
D = 4096
TOPK = 16
NW = 32
ROWS_PER_W = N_ROWS // NW
CH = 2
NCH = ROWS_PER_W // CH
INV_TEMP = 10.0
NEG = jnp.float32(-3.0e38)
CHUNKS_PER_ROW = D // 16


def _bf16_round(v):
    u = plsc.bitcast(v, jnp.int32)
    u = u + jnp.int32(0x8000)
    u = lax.bitwise_and(u, jnp.int32(-65536))
    return plsc.bitcast(u, jnp.float32)


def _rsqrt_vec(x):
    x = jnp.maximum(x, jnp.float32(1e-24))
    i = plsc.bitcast(x, jnp.int32)
    i = jnp.int32(0x5F3759DF) - lax.shift_right_logical(i, 1)
    g = plsc.bitcast(i, jnp.float32)
    for _ in range(3):
        g = g * (1.5 - 0.5 * x * g * g)
    return g


def _decode(p):
    c = lax.shift_right_logical(p, 6)
    r = lax.bitwise_and(lax.shift_right_logical(p, 1), 31)
    l = lax.shift_left(lax.bitwise_and(p, 1), 4)
    return c, r, l


_LANE = lambda: lax.iota(jnp.int32, 16)


def _extract(vec, q):
    return jnp.sum(jnp.where(_LANE() == q, vec, 0.0))


def _sc_sims_topk(cb, an):
    mesh = plsc.VectorSubcoreMesh(core_axis_name="c", subcore_axis_name="s")

    @functools.partial(
        pl.kernel,
        out_type=[
            jax.ShapeDtypeStruct((NW * TOPK,), jnp.float32),
            jax.ShapeDtypeStruct((NW * TOPK,), jnp.int32),
        ],
        mesh=mesh,
        scratch_types=[
            pltpu.VMEM((1, 4, 32, 32), jnp.float32),
            pltpu.VMEM((2, CH, 4, 32, 32), jnp.float32),
            pltpu.VMEM((ROWS_PER_W,), jnp.float32),
            pltpu.VMEM((TOPK,), jnp.float32),
            pltpu.VMEM((TOPK,), jnp.int32),
            pltpu.SemaphoreType.DMA,
            pltpu.SemaphoreType.DMA,
        ],
        compiler_params=pltpu.CompilerParams(needs_layout_passes=False),
    )
    def k(cb_hbm, an_hbm, vals_hbm, idx_hbm, an_v, bufs, sims_v, lv, li, sem0, sem1):
        cid = lax.axis_index("c")
        sid = lax.axis_index("s")
        wid = sid * 2 + cid
        base = wid * ROWS_PER_W

        pltpu.sync_copy(an_hbm, an_v)

        def a_nsq_step(p, acc):
            c, r, l = _decode(p)
            v = an_v[0, c, r, pl.ds(l, 16)]
            return acc + v * v

        a_acc = lax.fori_loop(0, CHUNKS_PER_ROW, a_nsq_step, jnp.zeros((16,), jnp.float32))
        a_nsq = jnp.sum(a_acc)
        arn = jnp.sum(jnp.where(_LANE() == 0, _rsqrt_vec(jnp.full((16,), a_nsq, jnp.float32)), 0.0))

        def a_round_step(p, _):
            c, r, l = _decode(p)
            an_v[0, c, r, pl.ds(l, 16)] = _bf16_round(an_v[0, c, r, pl.ds(l, 16)] * arn)
            return 0

        lax.fori_loop(0, CHUNKS_PER_ROW, a_round_step, 0)

        sems = (sem0, sem1)

        def start(t, b):
            pltpu.make_async_copy(
                cb_hbm.at[pl.ds(base + t * CH, CH)], bufs.at[b], sems[b]
            ).start()

        def wait(b):
            pltpu.make_async_copy(
                cb_hbm.at[pl.ds(0, CH)], bufs.at[b], sems[b]
            ).wait()

        def compute_chunk(b, simvec, lane_off):

            def pass1(p, accs):
                c, r, l = _decode(p)
                out = []
                for q in range(CH):
                    v = bufs[b, q, c, r, pl.ds(l, 16)]
                    out.append(accs[q] + v * v)
                return tuple(out)

            accs = lax.fori_loop(
                0, CHUNKS_PER_ROW, pass1,
                tuple(jnp.zeros((16,), jnp.float32) for _ in range(CH)),
            )
            nsqv = jnp.zeros((16,), jnp.float32)
            for q in range(CH):
                nsqv = jnp.where(_LANE() == q, jnp.sum(accs[q]), nsqv)
            rnv = _rsqrt_vec(nsqv)
            rns = [_extract(rnv, q) for q in range(CH)]

            def pass2(p, accs):
                c, r, l = _decode(p)
                van = an_v[0, c, r, pl.ds(l, 16)]
                out = []
                for q in range(CH):
                    vb = bufs[b, q, c, r, pl.ds(l, 16)] * rns[q]
                    out.append(accs[q] + _bf16_round(vb) * van)
                return tuple(out)

            daccs = lax.fori_loop(
                0, CHUNKS_PER_ROW, pass2,
                tuple(jnp.zeros((16,), jnp.float32) for _ in range(CH)),
            )
            for q in range(CH):
                simvec = jnp.where(
                    _LANE() == lane_off + q, jnp.sum(daccs[q]), simvec
                )
            return simvec

        start(0, 0)

        def ring(tt, simvec):
            t0 = tt * 2
            off = (lax.bitwise_and(tt, 3)) * (2 * CH)
            start(t0 + 1, 1)
            wait(0)
            simvec = compute_chunk(0, simvec, off)

            @pl.when(tt < NCH // 2 - 1)
            def _():
                start(t0 + 2, 0)

            wait(1)
            simvec = compute_chunk(1, simvec, off + CH)

            @pl.when(lax.bitwise_and(tt, 3) == 3)
            def _():
                g16 = lax.shift_right_logical(tt, 2)
                sims_v[pl.ds(g16 * 16, 16)] = simvec

            return jnp.where(
                lax.bitwise_and(tt, 3) == 3, jnp.zeros((16,), jnp.float32), simvec
            )

        lax.fori_loop(0, NCH // 2, ring, jnp.zeros((16,), jnp.float32))

        def topk_round(k_i, carry):
            valvec, idxvec = carry

            def fold(j, c2):
                bv, bi = c2
                chunk = sims_v[pl.ds(j * 16, 16)]
                cand_i = j * 16 + _LANE()
                upd = chunk > bv
                return jnp.where(upd, chunk, bv), jnp.where(upd, cand_i, bi)

            bv, bi = fold(0, (jnp.full((16,), NEG), jnp.zeros((16,), jnp.int32)))
            bv, bi = lax.fori_loop(1, ROWS_PER_W // 16, fold, (bv, bi))
            maxv = jnp.max(bv)
            pick = jnp.min(jnp.where(bv == maxv, bi, jnp.int32(2**30)))
            cc = lax.shift_right_logical(pick, 4)
            ll = lax.bitwise_and(pick, 15)
            chunk = sims_v[pl.ds(cc * 16, 16)]
            sims_v[pl.ds(cc * 16, 16)] = jnp.where(_LANE() == ll, NEG, chunk)
            valvec = jnp.where(_LANE() == k_i, maxv, valvec)
            idxvec = jnp.where(_LANE() == k_i, base + pick, idxvec)
            return valvec, idxvec

        valvec, idxvec = lax.fori_loop(
            0, TOPK, topk_round,
            (jnp.full((16,), NEG), jnp.zeros((16,), jnp.int32)),
        )
        lv[...] = valvec
        li[...] = idxvec
        pltpu.sync_copy(lv, vals_hbm.at[pl.ds(wid * TOPK, TOPK)])
        pltpu.sync_copy(li, idx_hbm.at[pl.ds(wid * TOPK, TOPK)])

    return k(cb, an)


def _sc_merge_gather(cb, vals, idxs):
    mesh = plsc.VectorSubcoreMesh(core_axis_name="c", subcore_axis_name="s")

    @functools.partial(
        pl.kernel,
        out_type=jax.ShapeDtypeStruct((D,), jnp.float32),
        mesh=mesh,
        scratch_types=[
            pltpu.VMEM((NW * TOPK,), jnp.float32),
            pltpu.VMEM((NW * TOPK,), jnp.int32),
            pltpu.SMEM((TOPK,), jnp.int32),
            pltpu.VMEM((4, 4, 32, 32), jnp.float32),
            pltpu.VMEM((D,), jnp.float32),
            pltpu.SemaphoreType.DMA,
            pltpu.SemaphoreType.DMA,
        ],
        compiler_params=pltpu.CompilerParams(needs_layout_passes=False),
    )
    def k(cb_hbm, vals_hbm, idx_hbm, out_hbm, vals_v, idx_v, idx_smem, rows_v, out_v, semi, semg):
        cid = lax.axis_index("c")
        sid = lax.axis_index("s")

        @pl.when((cid == 0) & (sid == 0))
        def _():
            pltpu.sync_copy(vals_hbm, vals_v)
            pltpu.sync_copy(idx_hbm, idx_v)

            def topk_round(k_i, valvec):
                def fold(j, c2):
                    bv, bi = c2
                    cv = vals_v[pl.ds(j * 16, 16)]
                    ci = idx_v[pl.ds(j * 16, 16)]
                    upd = cv > bv
                    return jnp.where(upd, cv, bv), jnp.where(upd, ci, bi)

                bv, bi = lax.fori_loop(
                    0, NW, fold,
                    (jnp.full((16,), NEG), jnp.zeros((16,), jnp.int32)),
                )
                maxv = jnp.max(bv)
                pick = jnp.min(jnp.where(bv == maxv, bi, jnp.int32(2**30)))
                idx_smem[k_i] = pick

                def mask(j, _):
                    cv = vals_v[pl.ds(j * 16, 16)]
                    ci = idx_v[pl.ds(j * 16, 16)]
                    vals_v[pl.ds(j * 16, 16)] = jnp.where(ci == pick, NEG, cv)
                    return 0

                lax.fori_loop(0, NW, mask, 0)
                return jnp.where(_LANE() == k_i, maxv, valvec)

            valvec = lax.fori_loop(0, TOPK, topk_round, jnp.full((16,), NEG))

            z = valvec * INV_TEMP
            z = z - jnp.max(z)
            e = jnp.exp(z)
            w = e / jnp.sum(e)
            wts = [_extract(w, q) for q in range(TOPK)]

            for b4 in range(4):
                copies = [
                    pltpu.make_async_copy(
                        cb_hbm.at[pl.ds(idx_smem[b4 * 4 + q], 1)],
                        rows_v.at[pl.ds(q, 1)],
                        semg,
                    )
                    for q in range(4)
                ]
                for c in copies:
                    c.start()
                for c in copies:
                    c.wait()

                def combine(p, _, b4=b4):
                    c, r, l = _decode(p)
                    acc = (
                        jnp.zeros((16,), jnp.float32)
                        if b4 == 0
                        else out_v[pl.ds(p * 16, 16)]
                    )
                    for q in range(4):
                        acc = acc + rows_v[q, c, r, pl.ds(l, 16)] * wts[b4 * 4 + q]
                    out_v[pl.ds(p * 16, 16)] = acc
                    return 0

                lax.fori_loop(0, CHUNKS_PER_ROW, combine, 0)
            pltpu.sync_copy(out_v, out_hbm)

    return k(cb, vals, idxs)


def kernel(codebook, anchor_noise):
    vals, idxs = _sc_sims_topk(codebook, anchor_noise)
    out = _sc_merge_gather(codebook, vals, idxs)
    return out.reshape(1, 4, 32, 32)


# speedup vs baseline: 1.0120x; 1.0120x over previous
import functools

import jax
import jax.numpy as jnp
from jax import lax
from jax.experimental import pallas as pl
from jax.experimental.pallas import tpu as pltpu
from jax.experimental.pallas import tpu_sc as plsc

N_ROWS = 8192
D = 4096
TOPK = 16
NW = 32
ROWS_PER_W = N_ROWS // NW
CH = 2
NCH = ROWS_PER_W // CH
INV_TEMP = 10.0
NEG = jnp.float32(-3.0e38)
CHUNKS_PER_ROW = D // 16


def _bf16_round(v):
    u = plsc.bitcast(v, jnp.int32)
    u = u + jnp.int32(0x8000)
    u = lax.bitwise_and(u, jnp.int32(-65536))
    return plsc.bitcast(u, jnp.float32)


def _rsqrt_vec(x):
    x = jnp.maximum(x, jnp.float32(1e-24))
    i = plsc.bitcast(x, jnp.int32)
    i = jnp.int32(0x5F3759DF) - lax.shift_right_logical(i, 1)
    g = plsc.bitcast(i, jnp.float32)
    for _ in range(3):
        g = g * (1.5 - 0.5 * x * g * g)
    return g


def _decode(p):
    c = lax.shift_right_logical(p, 6)
    r = lax.bitwise_and(lax.shift_right_logical(p, 1), 31)
    l = lax.shift_left(lax.bitwise_and(p, 1), 4)
    return c, r, l


_LANE = lambda: lax.iota(jnp.int32, 16)


def _extract(vec, q):
    return jnp.sum(jnp.where(_LANE() == q, vec, 0.0))


def _sc_sims_topk(cb, an):
    mesh = plsc.VectorSubcoreMesh(core_axis_name="c", subcore_axis_name="s")

    @functools.partial(
        pl.kernel,
        out_type=[
            jax.ShapeDtypeStruct((NW * TOPK,), jnp.float32),
            jax.ShapeDtypeStruct((NW * TOPK,), jnp.int32),
        ],
        mesh=mesh,
        scratch_types=[
            pltpu.VMEM((1, 4, 32, 32), jnp.float32),
            pltpu.VMEM((2, CH, 4, 32, 32), jnp.float32),
            pltpu.VMEM((ROWS_PER_W,), jnp.float32),
            pltpu.VMEM((TOPK,), jnp.float32),
            pltpu.VMEM((TOPK,), jnp.int32),
            pltpu.SemaphoreType.DMA,
            pltpu.SemaphoreType.DMA,
        ],
        compiler_params=pltpu.CompilerParams(needs_layout_passes=False),
    )
    def k(cb_hbm, an_hbm, vals_hbm, idx_hbm, an_v, bufs, sims_v, lv, li, sem0, sem1):
        cid = lax.axis_index("c")
        sid = lax.axis_index("s")
        wid = sid * 2 + cid
        base = wid * ROWS_PER_W

        pltpu.sync_copy(an_hbm, an_v)

        def a_nsq_step(p, acc):
            c, r, l = _decode(p)
            v = an_v[0, c, r, pl.ds(l, 16)]
            return acc + v * v

        a_acc = lax.fori_loop(0, CHUNKS_PER_ROW, a_nsq_step, jnp.zeros((16,), jnp.float32))
        a_nsq = jnp.sum(a_acc)
        arn = jnp.sum(jnp.where(_LANE() == 0, _rsqrt_vec(jnp.full((16,), a_nsq, jnp.float32)), 0.0))

        def a_round_step(p, _):
            c, r, l = _decode(p)
            an_v[0, c, r, pl.ds(l, 16)] = _bf16_round(an_v[0, c, r, pl.ds(l, 16)] * arn)
            return 0

        lax.fori_loop(0, CHUNKS_PER_ROW, a_round_step, 0)

        sems = (sem0, sem1)

        def start(t, b):
            pltpu.make_async_copy(
                cb_hbm.at[pl.ds(base + t * CH, CH)], bufs.at[b], sems[b]
            ).start()

        def wait(b):
            pltpu.make_async_copy(
                cb_hbm.at[pl.ds(0, CH)], bufs.at[b], sems[b]
            ).wait()

        def compute_chunk(b, simvec, lane_off):

            def _addrs(i):
                c0 = lax.shift_right_logical(i, 3)
                r0 = lax.bitwise_and(lax.shift_left(i, 2), 31)
                return c0, [r0 + k for k in range(4)]

            def pass1(i, accs):
                c0, rs = _addrs(i)
                out = list(accs)
                for j in range(8):
                    rr = rs[j >> 1]
                    l = (j & 1) * 16
                    for q in range(CH):
                        v = bufs[b, q, c0, rr, pl.ds(l, 16)]
                        out[q] = out[q] + v * v
                return tuple(out)

            accs = lax.fori_loop(
                0, CHUNKS_PER_ROW // 8, pass1,
                tuple(jnp.zeros((16,), jnp.float32) for _ in range(CH)),
            )
            nsqv = jnp.zeros((16,), jnp.float32)
            for q in range(CH):
                nsqv = jnp.where(_LANE() == q, jnp.sum(accs[q]), nsqv)
            rnv = _rsqrt_vec(nsqv)
            rns = [_extract(rnv, q) for q in range(CH)]

            def pass2(i, accs):
                c0, rs = _addrs(i)
                out = list(accs)
                for j in range(8):
                    rr = rs[j >> 1]
                    l = (j & 1) * 16
                    van = an_v[0, c0, rr, pl.ds(l, 16)]
                    for q in range(CH):
                        vb = bufs[b, q, c0, rr, pl.ds(l, 16)] * rns[q]
                        out[q] = out[q] + _bf16_round(vb) * van
                return tuple(out)

            daccs = lax.fori_loop(
                0, CHUNKS_PER_ROW // 8, pass2,
                tuple(jnp.zeros((16,), jnp.float32) for _ in range(CH)),
            )
            for q in range(CH):
                simvec = jnp.where(
                    _LANE() == lane_off + q, jnp.sum(daccs[q]), simvec
                )
            return simvec

        start(0, 0)

        def ring(tt, simvec):
            t0 = tt * 2
            off = (lax.bitwise_and(tt, 3)) * (2 * CH)
            start(t0 + 1, 1)
            wait(0)
            simvec = compute_chunk(0, simvec, off)

            @pl.when(tt < NCH // 2 - 1)
            def _():
                start(t0 + 2, 0)

            wait(1)
            simvec = compute_chunk(1, simvec, off + CH)

            @pl.when(lax.bitwise_and(tt, 3) == 3)
            def _():
                g16 = lax.shift_right_logical(tt, 2)
                sims_v[pl.ds(g16 * 16, 16)] = simvec

            return jnp.where(
                lax.bitwise_and(tt, 3) == 3, jnp.zeros((16,), jnp.float32), simvec
            )

        lax.fori_loop(0, NCH // 2, ring, jnp.zeros((16,), jnp.float32))

        def topk_round(k_i, carry):
            valvec, idxvec = carry

            def fold(j, c2):
                bv, bi = c2
                chunk = sims_v[pl.ds(j * 16, 16)]
                cand_i = j * 16 + _LANE()
                upd = chunk > bv
                return jnp.where(upd, chunk, bv), jnp.where(upd, cand_i, bi)

            bv, bi = fold(0, (jnp.full((16,), NEG), jnp.zeros((16,), jnp.int32)))
            bv, bi = lax.fori_loop(1, ROWS_PER_W // 16, fold, (bv, bi))
            maxv = jnp.max(bv)
            pick = jnp.min(jnp.where(bv == maxv, bi, jnp.int32(2**30)))
            cc = lax.shift_right_logical(pick, 4)
            ll = lax.bitwise_and(pick, 15)
            chunk = sims_v[pl.ds(cc * 16, 16)]
            sims_v[pl.ds(cc * 16, 16)] = jnp.where(_LANE() == ll, NEG, chunk)
            valvec = jnp.where(_LANE() == k_i, maxv, valvec)
            idxvec = jnp.where(_LANE() == k_i, base + pick, idxvec)
            return valvec, idxvec

        valvec, idxvec = lax.fori_loop(
            0, TOPK, topk_round,
            (jnp.full((16,), NEG), jnp.zeros((16,), jnp.int32)),
        )
        lv[...] = valvec
        li[...] = idxvec
        pltpu.sync_copy(lv, vals_hbm.at[pl.ds(wid * TOPK, TOPK)])
        pltpu.sync_copy(li, idx_hbm.at[pl.ds(wid * TOPK, TOPK)])

    return k(cb, an)


def _sc_merge_gather(cb, vals, idxs):
    mesh = plsc.VectorSubcoreMesh(core_axis_name="c", subcore_axis_name="s")

    @functools.partial(
        pl.kernel,
        out_type=jax.ShapeDtypeStruct((D,), jnp.float32),
        mesh=mesh,
        scratch_types=[
            pltpu.VMEM((NW * TOPK,), jnp.float32),
            pltpu.VMEM((NW * TOPK,), jnp.int32),
            pltpu.SMEM((TOPK,), jnp.int32),
            pltpu.VMEM((4, 4, 32, 32), jnp.float32),
            pltpu.VMEM((D,), jnp.float32),
            pltpu.SemaphoreType.DMA,
            pltpu.SemaphoreType.DMA,
        ],
        compiler_params=pltpu.CompilerParams(needs_layout_passes=False),
    )
    def k(cb_hbm, vals_hbm, idx_hbm, out_hbm, vals_v, idx_v, idx_smem, rows_v, out_v, semi, semg):
        cid = lax.axis_index("c")
        sid = lax.axis_index("s")

        @pl.when((cid == 0) & (sid == 0))
        def _():
            pltpu.sync_copy(vals_hbm, vals_v)
            pltpu.sync_copy(idx_hbm, idx_v)

            def topk_round(k_i, valvec):
                def fold(j, c2):
                    bv, bi = c2
                    cv = vals_v[pl.ds(j * 16, 16)]
                    ci = idx_v[pl.ds(j * 16, 16)]
                    upd = cv > bv
                    return jnp.where(upd, cv, bv), jnp.where(upd, ci, bi)

                bv, bi = lax.fori_loop(
                    0, NW, fold,
                    (jnp.full((16,), NEG), jnp.zeros((16,), jnp.int32)),
                )
                maxv = jnp.max(bv)
                pick = jnp.min(jnp.where(bv == maxv, bi, jnp.int32(2**30)))
                idx_smem[k_i] = pick

                def mask(j, _):
                    cv = vals_v[pl.ds(j * 16, 16)]
                    ci = idx_v[pl.ds(j * 16, 16)]
                    vals_v[pl.ds(j * 16, 16)] = jnp.where(ci == pick, NEG, cv)
                    return 0

                lax.fori_loop(0, NW, mask, 0)
                return jnp.where(_LANE() == k_i, maxv, valvec)

            valvec = lax.fori_loop(0, TOPK, topk_round, jnp.full((16,), NEG))

            z = valvec * INV_TEMP
            z = z - jnp.max(z)
            e = jnp.exp(z)
            w = e / jnp.sum(e)
            wts = [_extract(w, q) for q in range(TOPK)]

            for b4 in range(4):
                copies = [
                    pltpu.make_async_copy(
                        cb_hbm.at[pl.ds(idx_smem[b4 * 4 + q], 1)],
                        rows_v.at[pl.ds(q, 1)],
                        semg,
                    )
                    for q in range(4)
                ]
                for c in copies:
                    c.start()
                for c in copies:
                    c.wait()

                def combine(p, _, b4=b4):
                    c, r, l = _decode(p)
                    acc = (
                        jnp.zeros((16,), jnp.float32)
                        if b4 == 0
                        else out_v[pl.ds(p * 16, 16)]
                    )
                    for q in range(4):
                        acc = acc + rows_v[q, c, r, pl.ds(l, 16)] * wts[b4 * 4 + q]
                    out_v[pl.ds(p * 16, 16)] = acc
                    return 0

                lax.fori_loop(0, CHUNKS_PER_ROW, combine, 0)
            pltpu.sync_copy(out_v, out_hbm)

    return k(cb, vals, idxs)


def kernel(codebook, anchor_noise):
    vals, idxs = _sc_sims_topk(codebook, anchor_noise)
    out = _sc_merge_gather(codebook, vals, idxs)
    return out.reshape(1, 4, 32, 32)


# speedup vs baseline: 1.1301x; 1.1167x over previous
import functools

import jax
import jax.numpy as jnp
from jax import lax
from jax.experimental import pallas as pl
from jax.experimental.pallas import tpu as pltpu
from jax.experimental.pallas import tpu_sc as plsc

N_ROWS = 8192
D = 4096
TOPK = 16
BLOCK_ROWS = 512
N_BLOCKS = N_ROWS // BLOCK_ROWS
INV_TEMP = 10.0
CHUNKS_PER_ROW = D // 16

_LANE = lambda: lax.iota(jnp.int32, 16)


def _sims_topk_body(an_ref, cb_ref, w_ref, i_ref, sims_scr):
    pid = pl.program_id(0)
    blk = cb_ref[...]
    normsq = jnp.sum(blk * blk, axis=1)
    rnorm = 1.0 / jnp.maximum(jnp.sqrt(normsq), 1e-12)
    bn = blk * rnorm[:, None]
    a = an_ref[...]
    a_rnorm = 1.0 / jnp.maximum(jnp.sqrt(jnp.sum(a * a)), 1e-12)
    an = a * a_rnorm
    bn_r = bn.astype(jnp.bfloat16).astype(jnp.float32)
    an_r = an.astype(jnp.bfloat16).astype(jnp.float32)
    s = jnp.sum(bn_r * an_r, axis=1)
    sims_scr[pid, :] = s

    @pl.when(pid == N_BLOCKS - 1)
    def _finalize():
        sims = sims_scr[...]
        ids = (
            lax.broadcasted_iota(jnp.int32, sims.shape, 0) * BLOCK_ROWS
            + lax.broadcasted_iota(jnp.int32, sims.shape, 1)
        )

        def step(k, carry):
            sims_c, vals, idxs = carry
            m = jnp.max(sims_c)
            pick = jnp.min(jnp.where(sims_c == m, ids, jnp.int32(2**30)))
            lane = lax.broadcasted_iota(jnp.int32, (1, TOPK), 1)
            vals = jnp.where(lane == k, m, vals)
            idxs = jnp.where(lane == k, pick, idxs)
            sims_c = jnp.where(ids == pick, -jnp.inf, sims_c)
            return sims_c, vals, idxs

        init = (
            sims,
            jnp.zeros((1, TOPK), jnp.float32),
            jnp.zeros((1, TOPK), jnp.int32),
        )
        _, vals, idxs = lax.fori_loop(0, TOPK, step, init)
        z = vals * INV_TEMP
        z = z - jnp.max(z)
        e = jnp.exp(z)
        w_ref[...] = e / jnp.sum(e)
        i_ref[...] = idxs


def _sims_topk(cb, an):
    return pl.pallas_call(
        _sims_topk_body,
        grid=(N_BLOCKS,),
        in_specs=[
            pl.BlockSpec((1, D), lambda i: (0, 0)),
            pl.BlockSpec((BLOCK_ROWS, D), lambda i: (i, 0)),
        ],
        out_specs=[
            pl.BlockSpec((1, TOPK), lambda i: (0, 0)),
            pl.BlockSpec((1, TOPK), lambda i: (0, 0)),
        ],
        out_shape=[
            jax.ShapeDtypeStruct((1, TOPK), jnp.float32),
            jax.ShapeDtypeStruct((1, TOPK), jnp.int32),
        ],
        scratch_shapes=[pltpu.VMEM((N_BLOCKS, BLOCK_ROWS), jnp.float32)],
    )(an, cb)


def _sc_gather_combine(cb4d, idxs, weights):
    mesh = plsc.VectorSubcoreMesh(core_axis_name="c", subcore_axis_name="s")

    @functools.partial(
        pl.kernel,
        out_type=jax.ShapeDtypeStruct((D,), jnp.float32),
        mesh=mesh,
        scratch_types=[
            pltpu.VMEM((TOPK,), jnp.int32),
            pltpu.VMEM((TOPK,), jnp.float32),
            pltpu.VMEM((4, 4, 32, 32), jnp.float32),
            pltpu.VMEM((D,), jnp.float32),
            pltpu.SemaphoreType.DMA,
        ],
        compiler_params=pltpu.CompilerParams(needs_layout_passes=False),
    )
    def k(cb_hbm, idx_hbm, w_hbm, out_hbm, idx_v, w_v, rows_v, out_v, sem):
        cid = lax.axis_index("c")
        sid = lax.axis_index("s")

        @pl.when((cid == 0) & (sid == 0))
        def _():
            pltpu.sync_copy(idx_hbm, idx_v)
            pltpu.sync_copy(w_hbm, w_v)
            w_vec = w_v[...]
            i_vec = idx_v[...]
            wts = [
                jnp.sum(jnp.where(_LANE() == q, w_vec, 0.0)) for q in range(TOPK)
            ]
            rows = [
                jnp.sum(jnp.where(_LANE() == q, i_vec, 0)) for q in range(TOPK)
            ]

            for b4 in range(4):
                copies = [
                    pltpu.make_async_copy(
                        cb_hbm.at[pl.ds(rows[b4 * 4 + q], 1)],
                        rows_v.at[pl.ds(q, 1)],
                        sem,
                    )
                    for q in range(4)
                ]
                for c in copies:
                    c.start()
                for c in copies:
                    c.wait()

                def combine(p, _, b4=b4):
                    c = lax.shift_right_logical(p, 6)
                    r = lax.bitwise_and(lax.shift_right_logical(p, 1), 31)
                    l = lax.shift_left(lax.bitwise_and(p, 1), 4)
                    acc = (
                        jnp.zeros((16,), jnp.float32)
                        if b4 == 0
                        else out_v[pl.ds(p * 16, 16)]
                    )
                    for q in range(4):
                        acc = acc + rows_v[q, c, r, pl.ds(l, 16)] * wts[b4 * 4 + q]
                    out_v[pl.ds(p * 16, 16)] = acc
                    return 0

                lax.fori_loop(0, CHUNKS_PER_ROW, combine, 0)
            pltpu.sync_copy(out_v, out_hbm)

    return k(cb4d, idxs, weights)


def kernel(codebook, anchor_noise):
    cb = codebook.reshape(N_ROWS, D)
    an = anchor_noise.reshape(1, D)
    weights, idxs = _sims_topk(cb, an)
    out = _sc_gather_combine(codebook, idxs.reshape(TOPK), weights.reshape(TOPK))
    return out.reshape(1, 4, 32, 32)
